# Initial kernel scaffold; baseline (speedup 1.0000x reference)
import jax
import jax.numpy as jnp
from jax.experimental import pallas as pl


def kernel(feats, segment_ids, num_segments):
    raise NotImplementedError("write your pallas kernel here")


# speedup vs baseline: 4.4304x; 4.4304x over previous
import functools

import jax
import jax.numpy as jnp
from jax import lax
from jax.experimental import pallas as pl
from jax.experimental.pallas import tpu as pltpu
from jax.experimental.pallas import tpu_sc as plsc

NC = 2
NS = 16
LANES = 16

ROWS = 320000
D = 128
SEGS = 10000
DC = D // NC
BLK = 128
SUP = 4
SUP_ROWS = BLK * SUP
NSUP = ROWS // SUP_ROWS

ZBLK = 512
NZ = (SEGS + ZBLK - 1) // ZBLK


def _body(feats_hbm, ids_hbm, out_hbm, buf, idx0, idx1, idx2, idx3, acc):
    c = lax.axis_index("c")
    s = lax.axis_index("s")
    idx_bufs = [idx0, idx1, idx2, idx3]

    zeros16 = jnp.zeros((LANES,), jnp.float32)

    def zero_row(i, _):
        for k in range(DC // LANES):
            buf[i, pl.ds(k * LANES, LANES)] = zeros16
        return 0

    lax.fori_loop(0, SUP_ROWS, zero_row, 0)

    for z in range(NZ):
        nrows = min(ZBLK, SEGS - z * ZBLK)

        @pl.when(z % NS == s)
        def _():
            pltpu.sync_copy(buf.at[pl.ds(0, nrows), :],
                            acc.at[pl.ds(z * ZBLK, nrows), :])

    plsc.subcore_barrier()

    def step(it, _):
        sb = s + it * NS
        r0 = sb * SUP_ROWS
        pltpu.sync_copy(
            feats_hbm.at[pl.ds(r0, SUP_ROWS), pl.ds(c * DC, DC)], buf)
        for j in range(SUP):
            pltpu.sync_copy(ids_hbm.at[pl.ds(r0 + j * BLK, BLK)], idx_bufs[j])
        for j in range(SUP):
            pltpu.sync_copy(buf.at[pl.ds(j * BLK, BLK), :],
                            acc.at[idx_bufs[j]], add=True)
        return 0

    nblk = (NSUP - s + NS - 1) // NS
    lax.fori_loop(0, nblk, step, 0)

    plsc.subcore_barrier()

    for z in range(NZ):
        nrows = min(ZBLK, SEGS - z * ZBLK)

        @pl.when(z % NS == s)
        def _():
            pltpu.sync_copy(
                acc.at[pl.ds(z * ZBLK, nrows), :],
                out_hbm.at[pl.ds(z * ZBLK, nrows), pl.ds(c * DC, DC)])


@jax.jit
def _run(feats, segment_ids, num_segments):
    ids = jnp.minimum(segment_ids, num_segments - 1).astype(jnp.int32)
    mesh = plsc.VectorSubcoreMesh(core_axis_name="c", subcore_axis_name="s")
    grid_kernel = pl.kernel(
        _body,
        out_type=jax.ShapeDtypeStruct((SEGS, D), jnp.float32),
        mesh=mesh,
        scratch_types=[
            pltpu.VMEM((SUP_ROWS, DC), jnp.float32),
            pltpu.VMEM((BLK,), jnp.int32),
            pltpu.VMEM((BLK,), jnp.int32),
            pltpu.VMEM((BLK,), jnp.int32),
            pltpu.VMEM((BLK,), jnp.int32),
            pltpu.VMEM_SHARED((SEGS, DC), jnp.float32),
        ],
        compiler_params=pltpu.CompilerParams(use_tc_tiling_on_sc=False),
    )
    return grid_kernel(feats, ids)


def kernel(feats, segment_ids, num_segments):
    return _run(feats, segment_ids, num_segments)


# speedup vs baseline: 7.9828x; 1.8018x over previous
import jax
import jax.numpy as jnp
from jax import lax
from jax.experimental import pallas as pl
from jax.experimental.pallas import tpu as pltpu
from jax.experimental.pallas import tpu_sc as plsc

NC = 2
NS = 16
LANES = 16

ROWS = 320000
D = 128
SEGS = 10000
DC = D // NC
BLK = 128
SUP = 4
SUP_ROWS = BLK * SUP
NSUP = ROWS // SUP_ROWS
KMAX = (NSUP + NS - 1) // NS * NS

ZBLK = 512
NZ = (SEGS + ZBLK - 1) // ZBLK


def _body(feats_hbm, ids_hbm, out_hbm,
          buf0, buf1, i00, i01, i02, i03, i10, i11, i12, i13,
          acc, sem_l0, sem_l1, sem_s):
    c = lax.axis_index("c")
    s = lax.axis_index("s")
    bufs = (buf0, buf1)
    idxs = ((i00, i01, i02, i03), (i10, i11, i12, i13))
    sem_l = (sem_l0, sem_l1)

    def fire_load(k, slot):
        r0 = (s + k * NS) * SUP_ROWS
        pltpu.async_copy(
            feats_hbm.at[pl.ds(r0, SUP_ROWS), pl.ds(c * DC, DC)],
            bufs[slot], sem_l[slot])
        for j in range(SUP):
            pltpu.async_copy(ids_hbm.at[pl.ds(r0 + j * BLK, BLK)],
                             idxs[slot][j], sem_l[slot])

    def drain_load(slot):
        pltpu.make_async_copy(
            feats_hbm.at[pl.ds(0, SUP_ROWS), pl.ds(c * DC, DC)],
            bufs[slot], sem_l[slot]).wait()
        for j in range(SUP):
            pltpu.make_async_copy(ids_hbm.at[pl.ds(0, BLK)],
                                  idxs[slot][j], sem_l[slot]).wait()

    def valid(k):
        return (s + k * NS) < NSUP

    zeros16 = jnp.zeros((LANES,), jnp.float32)

    def zero_row(i, _):
        for t in range(DC // LANES):
            buf0[i, pl.ds(t * LANES, LANES)] = zeros16
        return 0

    lax.fori_loop(0, SUP_ROWS, zero_row, 0)

    for z in range(NZ):
        nrows = min(ZBLK, SEGS - z * ZBLK)

        @pl.when(z % NS == s)
        def _():
            pltpu.sync_copy(buf0.at[pl.ds(0, nrows), :],
                            acc.at[pl.ds(z * ZBLK, nrows), :])

    plsc.subcore_barrier()

    @pl.when(valid(0))
    def _():
        fire_load(0, 0)

    @pl.when(valid(1))
    def _():
        fire_load(1, 1)

    def step(it, _):
        for half in range(2):
            k = 2 * it + half

            @pl.when(valid(k))
            def _():
                drain_load(half)
                descs = [
                    pltpu.async_copy(
                        bufs[half].at[pl.ds(j * BLK, BLK), :],
                        acc.at[idxs[half][j]], sem_s, add=True)
                    for j in range(SUP)
                ]
                for d in descs:
                    d.wait()

                @pl.when(valid(k + 2))
                def _():
                    fire_load(k + 2, half)

        return 0

    lax.fori_loop(0, KMAX // NS // 2, step, 0)

    plsc.subcore_barrier()

    for z in range(NZ):
        nrows = min(ZBLK, SEGS - z * ZBLK)

        @pl.when(z % NS == s)
        def _():
            pltpu.sync_copy(
                acc.at[pl.ds(z * ZBLK, nrows), :],
                out_hbm.at[pl.ds(z * ZBLK, nrows), pl.ds(c * DC, DC)])


@jax.jit
def _run(feats, segment_ids, num_segments):
    ids = jnp.minimum(segment_ids, num_segments - 1).astype(jnp.int32)
    mesh = plsc.VectorSubcoreMesh(core_axis_name="c", subcore_axis_name="s")
    grid_kernel = pl.kernel(
        _body,
        out_type=jax.ShapeDtypeStruct((SEGS, D), jnp.float32),
        mesh=mesh,
        scratch_types=[
            pltpu.VMEM((SUP_ROWS, DC), jnp.float32),
            pltpu.VMEM((SUP_ROWS, DC), jnp.float32),
        ] + [pltpu.VMEM((BLK,), jnp.int32) for _ in range(2 * SUP)] + [
            pltpu.VMEM_SHARED((SEGS, DC), jnp.float32),
            pltpu.SemaphoreType.DMA,
            pltpu.SemaphoreType.DMA,
            pltpu.SemaphoreType.DMA,
        ],
        compiler_params=pltpu.CompilerParams(use_tc_tiling_on_sc=False),
    )
    return grid_kernel(feats, ids)


def kernel(feats, segment_ids, num_segments):
    return _run(feats, segment_ids, num_segments)


# speedup vs baseline: 8.3197x; 1.0422x over previous
import jax
import jax.numpy as jnp
from jax import lax
from jax.experimental import pallas as pl
from jax.experimental.pallas import tpu as pltpu
from jax.experimental.pallas import tpu_sc as plsc

NC = 2
NS = 16
LANES = 16
NBUF = 3

ROWS = 320000
D = 128
SEGS = 10000
DC = D // NC
BLK = 128
SUP = 2
SUP_ROWS = BLK * SUP
NSUP = ROWS // SUP_ROWS
KPT = (NSUP + NS - 1) // NS

ZBLK = 512
NZ = (SEGS + ZBLK - 1) // ZBLK


def _body(feats_hbm, ids_hbm, out_hbm, bufs, idxs, acc, sem_l, sem_s):
    c = lax.axis_index("c")
    s = lax.axis_index("s")

    def fire_load(k, slot):
        r0 = (s + k * NS) * SUP_ROWS
        pltpu.async_copy(
            feats_hbm.at[pl.ds(r0, SUP_ROWS), pl.ds(c * DC, DC)],
            bufs[slot], sem_l[slot])
        for j in range(SUP):
            pltpu.async_copy(ids_hbm.at[pl.ds(r0 + j * BLK, BLK)],
                             idxs[slot][j], sem_l[slot])

    def drain_load(slot):
        pltpu.make_async_copy(
            feats_hbm.at[pl.ds(0, SUP_ROWS), pl.ds(c * DC, DC)],
            bufs[slot], sem_l[slot]).wait()
        for j in range(SUP):
            pltpu.make_async_copy(ids_hbm.at[pl.ds(0, BLK)],
                                  idxs[slot][j], sem_l[slot]).wait()

    def fire_scatter(slot):
        for j in range(SUP):
            pltpu.async_copy(bufs[slot].at[pl.ds(j * BLK, BLK), :],
                             acc.at[idxs[slot][j]], sem_s, add=True)

    def drain_scatter(slot):
        for j in range(SUP):
            pltpu.make_async_copy(bufs[slot].at[pl.ds(j * BLK, BLK), :],
                                  acc.at[idxs[slot][j]], sem_s).wait()

    def valid(k):
        return (s + k * NS) < NSUP

    zeros16 = jnp.zeros((LANES,), jnp.float32)

    def zero_row(i, _):
        for t in range(DC // LANES):
            bufs[0][i, pl.ds(t * LANES, LANES)] = zeros16
        return 0

    lax.fori_loop(0, ZBLK, zero_row, 0)

    for z in range(NZ):
        nrows = min(ZBLK, SEGS - z * ZBLK)

        @pl.when(z % NS == s)
        def _():
            pltpu.sync_copy(bufs[0].at[pl.ds(0, nrows), :],
                            acc.at[pl.ds(z * ZBLK, nrows), :])

    plsc.subcore_barrier()

    @pl.when(valid(0))
    def _():
        fire_load(0, 0)

    @pl.when(valid(1))
    def _():
        fire_load(1, 1)

    def step(it, _):
        for r in range(NBUF):
            k = NBUF * it + r

            @pl.when(valid(k))
            def _():
                drain_load(r)
                fire_scatter(r)

                @pl.when(k >= 1)
                def _():
                    drain_scatter((r + NBUF - 1) % NBUF)

                @pl.when(valid(k + 2))
                def _():
                    fire_load(k + 2, (r + 2) % NBUF)

        return 0

    lax.fori_loop(0, (KPT + NBUF - 1) // NBUF, step, 0)

    drain_scatter(0)

    plsc.subcore_barrier()

    for z in range(NZ):
        nrows = min(ZBLK, SEGS - z * ZBLK)

        @pl.when(z % NS == s)
        def _():
            pltpu.sync_copy(
                acc.at[pl.ds(z * ZBLK, nrows), :],
                out_hbm.at[pl.ds(z * ZBLK, nrows), pl.ds(c * DC, DC)])


def _body_flat(feats_hbm, ids_hbm, out_hbm,
               b0, b1, b2,
               i00, i01, i10, i11, i20, i21,
               acc, sl0, sl1, sl2, sem_s):
    _body(feats_hbm, ids_hbm, out_hbm,
          (b0, b1, b2),
          ((i00, i01), (i10, i11), (i20, i21)),
          acc, (sl0, sl1, sl2), sem_s)


@jax.jit
def _run(feats, segment_ids, num_segments):
    del num_segments
    ids = segment_ids.astype(jnp.int32)
    mesh = plsc.VectorSubcoreMesh(core_axis_name="c", subcore_axis_name="s")
    grid_kernel = pl.kernel(
        _body_flat,
        out_type=jax.ShapeDtypeStruct((SEGS, D), jnp.float32),
        mesh=mesh,
        scratch_types=[
            pltpu.VMEM((SUP_ROWS, DC), jnp.float32) for _ in range(NBUF)
        ] + [pltpu.VMEM((BLK,), jnp.int32) for _ in range(NBUF * SUP)] + [
            pltpu.VMEM_SHARED((SEGS, DC), jnp.float32),
            pltpu.SemaphoreType.DMA,
            pltpu.SemaphoreType.DMA,
            pltpu.SemaphoreType.DMA,
            pltpu.SemaphoreType.DMA,
        ],
        compiler_params=pltpu.CompilerParams(use_tc_tiling_on_sc=False),
    )
    return grid_kernel(feats, ids)


def kernel(feats, segment_ids, num_segments):
    return _run(feats, segment_ids, num_segments)
